# baseline (speedup 1.0000x reference)
import functools

import jax
import jax.numpy as jnp
from jax import lax
from jax.experimental import pallas as pl
from jax.experimental.pallas import tpu as pltpu
from jax.experimental.pallas import tpu_sc as plsc

_VOCAB = 1000000
_MAXLEN = 200
_EMBED = 32
_BATCH = 4096

_NC = 2
_NS = 16
_NW = _NC * _NS

_N = _BATCH * _MAXLEN
_PER_W = _N // _NW
_CHUNK = _MAXLEN
_NCHUNK = _PER_W // _CHUNK
_NBUF = 4
_NGRP = _NCHUNK // _NBUF


def _tpe(xf, token_table, pos_table):
    mesh = plsc.VectorSubcoreMesh(core_axis_name="c", subcore_axis_name="s")

    scratch = (
        [pltpu.VMEM((_CHUNK, _EMBED), jnp.float32) for _ in range(_NBUF)]
        + [pltpu.VMEM((_CHUNK, _EMBED), jnp.float32) for _ in range(_NBUF)]
        + [
            pltpu.VMEM((_PER_W,), jnp.int32),
            pltpu.VMEM((_MAXLEN, _EMBED), jnp.float32),
        ]
        + [pltpu.SemaphoreType.DMA for _ in range(2 * _NBUF)]
    )

    @functools.partial(
        pl.kernel,
        out_type=jax.ShapeDtypeStruct((_N, _EMBED), jnp.float32),
        mesh=mesh,
        compiler_params=pltpu.CompilerParams(use_tc_tiling_on_sc=False),
        scratch_types=scratch,
    )
    def k(x_hbm, tok_hbm, pos_hbm, out_hbm, *sc):
        gbuf = sc[0:_NBUF]
        obuf = sc[_NBUF:2 * _NBUF]
        idx_v = sc[2 * _NBUF]
        pos_v = sc[2 * _NBUF + 1]
        gsem = sc[2 * _NBUF + 2:3 * _NBUF + 2]
        osem = sc[3 * _NBUF + 2:4 * _NBUF + 2]

        wid = lax.axis_index("s") * _NC + lax.axis_index("c")
        base0 = wid * _PER_W
        pltpu.sync_copy(pos_hbm, pos_v)
        pltpu.sync_copy(x_hbm.at[pl.ds(base0, _PER_W)], idx_v)

        def fire_gather(c, b):
            pltpu.async_copy(
                tok_hbm.at[idx_v.at[pl.ds(c * _CHUNK, _CHUNK)]], gbuf[b], gsem[b]
            )

        def wait_gather(b):
            pltpu.make_async_copy(
                out_hbm.at[pl.ds(0, _CHUNK)], gbuf[b], gsem[b]
            ).wait()

        def add_pos(b):
            def m_body(m, carry):
                obuf[b][m, 0:16] = gbuf[b][m, 0:16] + pos_v[m, 0:16]
                obuf[b][m, 16:32] = gbuf[b][m, 16:32] + pos_v[m, 16:32]
                return carry

            lax.fori_loop(0, _MAXLEN, m_body, 0, unroll=8)

        def fire_out(c, b):
            pltpu.async_copy(
                obuf[b], out_hbm.at[pl.ds(base0 + c * _CHUNK, _CHUNK)], osem[b]
            )

        def wait_out(b):
            pltpu.make_async_copy(
                obuf[b], out_hbm.at[pl.ds(0, _CHUNK)], osem[b]
            ).wait()

        for b in range(_NBUF):
            fire_gather(b, b)
        for b in range(_NBUF):
            wait_gather(b)
            add_pos(b)
            fire_out(b, b)
            fire_gather(_NBUF + b, b)

        def group(cc, carry):
            c0 = cc * _NBUF
            for b in range(_NBUF):
                wait_gather(b)
                wait_out(b)
                add_pos(b)
                fire_out(c0 + b, b)
                fire_gather(c0 + _NBUF + b, b)
            return carry

        lax.fori_loop(1, _NGRP - 1, group, 0)

        c0 = (_NGRP - 1) * _NBUF
        for b in range(_NBUF):
            wait_gather(b)
            wait_out(b)
            add_pos(b)
            fire_out(c0 + b, b)
        for b in range(_NBUF):
            wait_out(b)

    return k(xf, token_table, pos_table)


def kernel(x, token_table, pos_table):
    xf = x.reshape(-1).astype(jnp.int32)
    out = _tpe(xf, token_table, pos_table)
    return out.reshape(x.shape[0], x.shape[1], _EMBED)


# speedup vs baseline: 1.1927x; 1.1927x over previous
import functools

import jax
import jax.numpy as jnp
from jax import lax
from jax.experimental import pallas as pl
from jax.experimental.pallas import tpu as pltpu
from jax.experimental.pallas import tpu_sc as plsc

_VOCAB = 1000000
_MAXLEN = 200
_EMBED = 32
_BATCH = 4096

_NC = 2
_NS = 16
_NW = _NC * _NS

_N = _BATCH * _MAXLEN
_PER_W = _N // _NW
_SEQS_PER_CHUNK = 4
_CHUNK = _SEQS_PER_CHUNK * _MAXLEN
_NCHUNK = _PER_W // _CHUNK
_NBUF = 4
_NGRP = _NCHUNK // _NBUF


def _tpe(xf, token_table, pos_table):
    mesh = plsc.VectorSubcoreMesh(core_axis_name="c", subcore_axis_name="s")

    scratch = (
        [pltpu.VMEM((_CHUNK,), jnp.int32) for _ in range(_NBUF)]
        + [pltpu.VMEM((_CHUNK, _EMBED), jnp.float32) for _ in range(_NBUF)]
        + [pltpu.VMEM((_MAXLEN, _EMBED), jnp.float32)]
        + [pltpu.SemaphoreType.DMA for _ in range(3 * _NBUF)]
    )

    @functools.partial(
        pl.kernel,
        out_type=jax.ShapeDtypeStruct((_N, _EMBED), jnp.float32),
        mesh=mesh,
        compiler_params=pltpu.CompilerParams(use_tc_tiling_on_sc=False),
        scratch_types=scratch,
    )
    def k(x_hbm, tok_hbm, pos_hbm, out_hbm, *sc):
        ibuf = sc[0:_NBUF]
        gbuf = sc[_NBUF:2 * _NBUF]
        pos_v = sc[2 * _NBUF]
        isem = sc[2 * _NBUF + 1:3 * _NBUF + 1]
        gsem = sc[3 * _NBUF + 1:4 * _NBUF + 1]
        osem = sc[4 * _NBUF + 1:5 * _NBUF + 1]

        wid = lax.axis_index("s") * _NC + lax.axis_index("c")
        base0 = wid * _PER_W
        pltpu.sync_copy(pos_hbm, pos_v)

        def fire_idx(c, b):
            pltpu.async_copy(
                x_hbm.at[pl.ds(base0 + c * _CHUNK, _CHUNK)], ibuf[b], isem[b]
            )

        def wait_idx(b):
            pltpu.make_async_copy(
                x_hbm.at[pl.ds(0, _CHUNK)], ibuf[b], isem[b]
            ).wait()

        def fire_gather(c, b):
            pltpu.async_copy(tok_hbm.at[ibuf[b]], gbuf[b], gsem[b])

        def wait_gather(b):
            pltpu.make_async_copy(
                out_hbm.at[pl.ds(0, _CHUNK)], gbuf[b], gsem[b]
            ).wait()

        def add_pos(b):
            def m_body(m, carry):
                p0 = pos_v[m, 0:16]
                p1 = pos_v[m, 16:32]
                for s in range(_SEQS_PER_CHUNK):
                    r = s * _MAXLEN + m
                    plsc.addupdate(gbuf[b].at[r, pl.ds(0, 16)], p0)
                    plsc.addupdate(gbuf[b].at[r, pl.ds(16, 16)], p1)
                return carry

            lax.fori_loop(0, _MAXLEN, m_body, 0, unroll=4)

        def fire_out(c, b):
            pltpu.async_copy(
                gbuf[b], out_hbm.at[pl.ds(base0 + c * _CHUNK, _CHUNK)], osem[b]
            )

        def wait_out(b):
            pltpu.make_async_copy(
                gbuf[b], out_hbm.at[pl.ds(0, _CHUNK)], osem[b]
            ).wait()

        def step(c, b, first, last):
            b1 = (b + 1) % _NBUF
            wait_gather(b)
            if not last:
                wait_idx(b1)
                if not (first and b < _NBUF - 1):
                    wait_out(b1)
                fire_gather(c + 1, b1)
            add_pos(b)
            fire_out(c, b)
            if not last:
                fire_idx(c + _NBUF, b)

        for b in range(_NBUF):
            fire_idx(b, b)
        wait_idx(0)
        fire_gather(0, 0)

        for b in range(_NBUF):
            step(b, b, first=True, last=False)

        def group(cc, carry):
            c0 = cc * _NBUF
            for b in range(_NBUF):
                step(c0 + b, b, first=False, last=False)
            return carry

        lax.fori_loop(1, _NGRP - 1, group, 0)

        c0 = (_NGRP - 1) * _NBUF
        for b in range(_NBUF):
            b1 = (b + 1) % _NBUF
            wait_gather(b)
            if b < _NBUF - 1:
                wait_idx(b1)
                wait_out(b1)
                fire_gather(c0 + b + 1, b1)
            add_pos(b)
            fire_out(c0 + b, b)
        for b in range(_NBUF):
            wait_out(b)

    return k(xf, token_table, pos_table)


def kernel(x, token_table, pos_table):
    xf = x.reshape(-1).astype(jnp.int32)
    out = _tpe(xf, token_table, pos_table)
    return out.reshape(x.shape[0], x.shape[1], _EMBED)
